# baseline (speedup 1.0000x reference)
import functools

import jax
import jax.numpy as jnp
from jax import lax
from jax.experimental import pallas as pl
from jax.experimental.pallas import tpu as pltpu
from jax.experimental.pallas import tpu_sc as plsc

NUM_EMB = 1000000
B = 16384
L = 50
D = 32
NC = 2
NS = 16
NW = NC * NS
BPW = B // NW
CH = 2
NDESC = BPW // CH
NBUF = 8


def _make_sc_kernel():
    mesh = plsc.VectorSubcoreMesh(core_axis_name="c", subcore_axis_name="s")

    @functools.partial(
        pl.kernel,
        mesh=mesh,
        out_type=jax.ShapeDtypeStruct((B, D), jnp.float32),
        compiler_params=pltpu.CompilerParams(use_tc_tiling_on_sc=False),
        scratch_types=[
            pltpu.VMEM((NDESC, CH * L), jnp.int32),
            pltpu.VMEM((NBUF, CH * L, D), jnp.float32),
            pltpu.VMEM((BPW, D), jnp.float32),
            pltpu.SemaphoreType.DMA((NBUF,)),
        ],
    )
    def k(idx_hbm, table_hbm, out_hbm, idx_v, buf_v, out_v, sems):
        wid = lax.axis_index("s") * NC + lax.axis_index("c")
        base = wid * BPW
        pltpu.sync_copy(idx_hbm.at[pl.ds(wid * NDESC, NDESC)], idx_v)

        for b in range(NBUF):
            pltpu.async_copy(table_hbm.at[idx_v.at[b]], buf_v.at[b], sems.at[b])

        def group_body(g, carry):
            for b in range(NBUF):
                t = g * NBUF + b
                pltpu.make_async_copy(
                    table_hbm.at[idx_v.at[0]], buf_v.at[b], sems.at[b]
                ).wait()
                for c in range(CH):
                    a0 = buf_v[b, c * L, pl.ds(0, 16)]
                    a1 = buf_v[b, c * L, pl.ds(16, 16)]
                    for j in range(1, L):
                        a0 = a0 + buf_v[b, c * L + j, pl.ds(0, 16)]
                        a1 = a1 + buf_v[b, c * L + j, pl.ds(16, 16)]
                    out_v[t * CH + c, pl.ds(0, 16)] = a0
                    out_v[t * CH + c, pl.ds(16, 16)] = a1
                nxt = t + NBUF

                @pl.when(nxt < NDESC)
                def _():
                    pltpu.async_copy(
                        table_hbm.at[idx_v.at[nxt]], buf_v.at[b], sems.at[b]
                    )

            return carry

        lax.fori_loop(0, NDESC // NBUF, group_body, 0)
        pltpu.sync_copy(out_v, out_hbm.at[pl.ds(base, BPW)])

    return k


_sc_kernel = _make_sc_kernel()


def kernel(x, table):
    idx = x.astype(jnp.int32).reshape(NW * NDESC, CH * L)
    return _sc_kernel(idx, table)


# speedup vs baseline: 1.0306x; 1.0306x over previous
import functools

import jax
import jax.numpy as jnp
from jax import lax
from jax.experimental import pallas as pl
from jax.experimental.pallas import tpu as pltpu
from jax.experimental.pallas import tpu_sc as plsc

NUM_EMB = 1000000
B = 16384
L = 50
D = 32
NC = 2
NS = 16
NW = NC * NS
BPW = B // NW
NBUF = 8

NTILE = NUM_EMB // 8
NDROW = NUM_EMB * D // 1024
JG = 8


def _make_detile_kernel():
    mesh = plsc.VectorSubcoreMesh(core_axis_name="c", subcore_axis_name="s")

    @functools.partial(
        pl.kernel,
        mesh=mesh,
        out_type=jax.ShapeDtypeStruct((NDROW, 8, 128), jnp.float32),
        scratch_types=[
            pltpu.VMEM((4 * JG, 8, D), jnp.float32),
            pltpu.VMEM((JG, 8, 128), jnp.float32),
        ],
    )
    def k(src_hbm, dst_hbm, vbuf, dbuf):
        wid = lax.axis_index("s") * NC + lax.axis_index("c")
        jlo = wid * NDROW // NW
        jhi = (wid + 1) * NDROW // NW
        nfull = (jhi - jlo) // JG

        def compact_row(jj, dj, tile0):
            for sp in range(8):
                for lg in range(8):
                    r = 32 * jj + 4 * sp + lg // 2
                    c0 = (lg % 2) * 16
                    dbuf[dj, sp, pl.ds(lg * 16, 16)] = vbuf[
                        tile0 + r // 8, r % 8, pl.ds(c0, 16)
                    ]

        def chunk_body(g, carry):
            j0 = jlo + g * JG
            pltpu.sync_copy(src_hbm.at[pl.ds(4 * j0, 4 * JG)], vbuf)
            for jj in range(JG):
                compact_row(jj, jj, 0)
            pltpu.sync_copy(dbuf, dst_hbm.at[pl.ds(j0, JG)])
            return carry

        lax.fori_loop(0, nfull, chunk_body, 0)

        def tail_body(j, carry):
            pltpu.sync_copy(src_hbm.at[pl.ds(4 * j, 4)], vbuf.at[pl.ds(0, 4)])
            compact_row(0, 0, 0)
            pltpu.sync_copy(dbuf.at[pl.ds(0, 1)], dst_hbm.at[pl.ds(j, 1)])
            return carry

        lax.fori_loop(jlo + nfull * JG, jhi, tail_body, 0)

    return k


def _make_gather_kernel():
    mesh = plsc.VectorSubcoreMesh(core_axis_name="c", subcore_axis_name="s")

    @functools.partial(
        pl.kernel,
        mesh=mesh,
        out_type=jax.ShapeDtypeStruct((B, D), jnp.float32),
        compiler_params=pltpu.CompilerParams(use_tc_tiling_on_sc=False),
        scratch_types=[
            pltpu.VMEM((BPW, L), jnp.int32),
            pltpu.VMEM((NBUF, L, D), jnp.float32),
            pltpu.VMEM((BPW, D), jnp.float32),
            pltpu.SemaphoreType.DMA((NBUF,)),
        ],
    )
    def k(idx_hbm, table_hbm, out_hbm, idx_v, buf_v, out_v, sems):
        wid = lax.axis_index("s") * NC + lax.axis_index("c")
        base = wid * BPW
        pltpu.sync_copy(idx_hbm.at[pl.ds(base, BPW)], idx_v)

        for b in range(NBUF):
            pltpu.async_copy(table_hbm.at[idx_v.at[b]], buf_v.at[b], sems.at[b])

        def group_body(g, carry):
            for b in range(NBUF):
                s = g * NBUF + b
                pltpu.make_async_copy(
                    table_hbm.at[idx_v.at[0]], buf_v.at[b], sems.at[b]
                ).wait()
                a0 = buf_v[b, 0, pl.ds(0, 16)]
                a1 = buf_v[b, 0, pl.ds(16, 16)]
                for j in range(1, L):
                    a0 = a0 + buf_v[b, j, pl.ds(0, 16)]
                    a1 = a1 + buf_v[b, j, pl.ds(16, 16)]
                out_v[s, pl.ds(0, 16)] = a0
                out_v[s, pl.ds(16, 16)] = a1
                nxt = s + NBUF

                @pl.when(nxt < BPW)
                def _():
                    pltpu.async_copy(
                        table_hbm.at[idx_v.at[nxt]], buf_v.at[b], sems.at[b]
                    )

            return carry

        lax.fori_loop(0, BPW // NBUF, group_body, 0)
        pltpu.sync_copy(out_v, out_hbm.at[pl.ds(base, BPW)])

    return k


_detile = _make_detile_kernel()
_gather = _make_gather_kernel()


def kernel(x, table):
    t3 = table.reshape(NTILE, 8, D)
    lin = _detile(t3)
    tbl = lin.reshape(NUM_EMB, D)
    idx = x.astype(jnp.int32)
    return _gather(idx, tbl)
